# initial kernel scaffold (unmeasured)
import jax
import jax.numpy as jnp
from jax import lax
from jax.experimental import pallas as pl
from jax.experimental.pallas import tpu as pltpu

N_DEV = 32
BM = 128
COMM_DTYPE = jnp.float8_e4m3fn


def kernel(x, w_mat, scale_x, scale_w):
    m_total, k_shard = x.shape
    _, n = w_mat.shape

    def body(x_ref, w_hbm, sx_ref, sw_ref, out_ref,
             x8_ref, xg_ref, wblk_ref, wsems, send_sems, recv_sems):
        my = lax.axis_index("i")

        x8_ref[...] = x_ref[...].astype(COMM_DTYPE)

        xg_ref[pl.ds(my * BM, BM), :] = x8_ref[pl.ds(my * BM, BM), :]

        sends = []
        for off in range(1, N_DEV):
            peer = lax.rem(my + off, N_DEV)
            rdma = pltpu.make_async_remote_copy(
                src_ref=x8_ref.at[pl.ds(peer * BM, BM), :],
                dst_ref=xg_ref.at[pl.ds(my * BM, BM), :],
                send_sem=send_sems.at[off],
                recv_sem=recv_sems.at[off],
                device_id=(peer,),
                device_id_type=pl.DeviceIdType.MESH,
            )
            rdma.start()
            sends.append(rdma)

        def start_w(d):
            s = lax.rem(my - d + N_DEV, N_DEV)
            cp = pltpu.make_async_copy(
                w_hbm.at[pl.ds(s * BM, BM), :],
                wblk_ref.at[d % 2],
                wsems.at[d % 2],
            )
            cp.start()
            return cp

        w_descs = {0: start_w(0), 1: start_w(1)}

        for d in range(N_DEV):
            s = lax.rem(my - d + N_DEV, N_DEV)
            if d > 0:
                recv = pltpu.make_async_remote_copy(
                    src_ref=x8_ref.at[pl.ds(0, BM), :],
                    dst_ref=xg_ref.at[pl.ds(s * BM, BM), :],
                    send_sem=send_sems.at[d],
                    recv_sem=recv_sems.at[d],
                    device_id=(0,),
                    device_id_type=pl.DeviceIdType.MESH,
                )
                recv.wait_recv()
            w_descs[d % 2].wait()
            a = xg_ref[pl.ds(s * BM, BM), :]
            wb = wblk_ref[d % 2].astype(COMM_DTYPE)
            contrib = jnp.dot(a, wb, preferred_element_type=jnp.float32)
            if d == 0:
                out_ref[...] = contrib
            else:
                out_ref[...] += contrib
            if d + 2 < N_DEV:
                w_descs[d % 2] = start_w(d + 2)

        scale = sx_ref[0] * sw_ref[0]
        out_ref[...] = jnp.maximum(out_ref[...] * scale, 0.0)

        for r in sends:
            r.wait_send()

    return pl.pallas_call(
        body,
        out_shape=jax.ShapeDtypeStruct((BM, n), jnp.float32),
        in_specs=[
            pl.BlockSpec(memory_space=pltpu.VMEM),
            pl.BlockSpec(memory_space=pltpu.ANY),
            pl.BlockSpec(memory_space=pltpu.SMEM),
            pl.BlockSpec(memory_space=pltpu.SMEM),
        ],
        out_specs=pl.BlockSpec(memory_space=pltpu.VMEM),
        scratch_shapes=[
            pltpu.VMEM((m_total, k_shard), COMM_DTYPE),
            pltpu.VMEM((N_DEV * BM, k_shard), COMM_DTYPE),
            pltpu.VMEM((2, BM, n), jnp.float32),
            pltpu.SemaphoreType.DMA((2,)),
            pltpu.SemaphoreType.DMA((N_DEV,)),
            pltpu.SemaphoreType.DMA((N_DEV,)),
        ],
    )(x, w_mat, scale_x, scale_w)


# baseline (device time: 64399 ns/iter reference)
import jax
import jax.numpy as jnp
from jax import lax
from jax.experimental import pallas as pl
from jax.experimental.pallas import tpu as pltpu

N_DEV = 32
BM = 128
COMM_DTYPE = jnp.float8_e4m3fn


def kernel(x, w_mat, scale_x, scale_w):
    m_total, k_shard = x.shape
    _, n = w_mat.shape

    def body(x_ref, w_hbm, sx_ref, sw_ref, out_ref,
             x8_ref, xg_ref, wblk_ref, wsems, send_sems, recv_sems):
        my = lax.axis_index("i")

        x8_ref[...] = x_ref[...].astype(COMM_DTYPE)

        xg_ref[pl.ds(my * BM, BM), :] = x8_ref[pl.ds(my * BM, BM), :]

        sends = []
        for off in range(1, N_DEV):
            peer = lax.rem(my + off, N_DEV)
            rdma = pltpu.make_async_remote_copy(
                src_ref=x8_ref.at[pl.ds(peer * BM, BM), :],
                dst_ref=xg_ref.at[pl.ds(my * BM, BM), :],
                send_sem=send_sems.at[off],
                recv_sem=recv_sems.at[off],
                device_id=(peer,),
                device_id_type=pl.DeviceIdType.MESH,
            )
            rdma.start()
            sends.append(rdma)

        def start_w(d):
            s = lax.rem(my - d + N_DEV, N_DEV)
            cp = pltpu.make_async_copy(
                w_hbm.at[pl.ds(s * BM, BM), :],
                wblk_ref.at[d % 2],
                wsems.at[d % 2],
            )
            cp.start()
            return cp

        w_descs = {0: start_w(0), 1: start_w(1)}

        for d in range(N_DEV):
            s = lax.rem(my - d + N_DEV, N_DEV)
            if d > 0:
                recv = pltpu.make_async_remote_copy(
                    src_ref=x8_ref.at[pl.ds(0, BM), :],
                    dst_ref=xg_ref.at[pl.ds(s * BM, BM), :],
                    send_sem=send_sems.at[d],
                    recv_sem=recv_sems.at[d],
                    device_id=(0,),
                    device_id_type=pl.DeviceIdType.MESH,
                )
                recv.wait_recv()
            w_descs[d % 2].wait()
            a = xg_ref[pl.ds(s * BM, BM), :]
            wb = wblk_ref[d % 2].astype(COMM_DTYPE)
            contrib = jnp.dot(a, wb, preferred_element_type=jnp.float32)
            if d == 0:
                out_ref[...] = contrib
            else:
                out_ref[...] += contrib
            if d + 2 < N_DEV:
                w_descs[d % 2] = start_w(d + 2)

        scale = sx_ref[0] * sw_ref[0]
        out_ref[...] = jnp.maximum(out_ref[...] * scale, 0.0)

        for r in sends:
            r.wait_send()

    return pl.pallas_call(
        body,
        out_shape=jax.ShapeDtypeStruct((BM, n), jnp.float32),
        in_specs=[
            pl.BlockSpec(memory_space=pltpu.VMEM),
            pl.BlockSpec(memory_space=pltpu.MemorySpace.HBM),
            pl.BlockSpec(memory_space=pltpu.SMEM),
            pl.BlockSpec(memory_space=pltpu.SMEM),
        ],
        out_specs=pl.BlockSpec(memory_space=pltpu.VMEM),
        scratch_shapes=[
            pltpu.VMEM((m_total, k_shard), COMM_DTYPE),
            pltpu.VMEM((N_DEV * BM, k_shard), COMM_DTYPE),
            pltpu.VMEM((2, BM, n), jnp.float32),
            pltpu.SemaphoreType.DMA((2,)),
            pltpu.SemaphoreType.DMA((N_DEV,)),
            pltpu.SemaphoreType.DMA((N_DEV,)),
        ],
    )(x, w_mat, scale_x, scale_w)


# device time: 63039 ns/iter; 1.0216x vs baseline; 1.0216x over previous
import jax
import jax.numpy as jnp
from jax import lax
from jax.experimental import pallas as pl
from jax.experimental.pallas import tpu as pltpu

N_DEV = 32
BM = 128
KC = 512
COMM_DTYPE = jnp.float8_e4m3fn


def kernel(x, w_mat, scale_x, scale_w):
    m_total, k_shard = x.shape
    k_total, n = w_mat.shape
    n_chunks = k_total // KC
    blocks_per_chunk = KC // BM

    def body(x_ref, w_hbm, sx_ref, sw_ref, out_ref,
             x8_ref, xg_ref, wblk_ref, wsems, send_sems, recv_sems):
        my = lax.axis_index("i")

        x8_ref[...] = x_ref[...].astype(COMM_DTYPE)

        xg_ref[:, pl.ds(my * BM, BM)] = x8_ref[pl.ds(my * BM, BM), :]

        sends = []
        for off in range(1, N_DEV):
            peer = lax.rem(my + off, N_DEV)
            rdma = pltpu.make_async_remote_copy(
                src_ref=x8_ref.at[pl.ds(peer * BM, BM), :],
                dst_ref=xg_ref.at[:, pl.ds(my * BM, BM)],
                send_sem=send_sems.at[off],
                recv_sem=recv_sems.at[off],
                device_id=(peer,),
                device_id_type=pl.DeviceIdType.MESH,
            )
            rdma.start()
            sends.append(rdma)

        def start_w(c):
            cp = pltpu.make_async_copy(
                w_hbm.at[pl.ds(c * KC, KC), :],
                wblk_ref.at[c % 2],
                wsems.at[c % 2],
            )
            cp.start()
            return cp

        w_descs = {0: start_w(0), 1: start_w(1)}

        for off in range(1, N_DEV):
            s = lax.rem(my - off + N_DEV, N_DEV)
            recv = pltpu.make_async_remote_copy(
                src_ref=x8_ref.at[pl.ds(0, BM), :],
                dst_ref=xg_ref.at[:, pl.ds(s * BM, BM)],
                send_sem=send_sems.at[off],
                recv_sem=recv_sems.at[off],
                device_id=(0,),
                device_id_type=pl.DeviceIdType.MESH,
            )
            recv.wait_recv()

        for c in range(n_chunks):
            w_descs[c % 2].wait()
            a = xg_ref[:, pl.ds(c * KC, KC)]
            wb = wblk_ref[c % 2].astype(COMM_DTYPE)
            contrib = jnp.dot(a, wb, preferred_element_type=jnp.float32)
            if c == 0:
                out_ref[...] = contrib
            else:
                out_ref[...] += contrib
            if c + 2 < n_chunks:
                w_descs[c % 2] = start_w(c + 2)

        scale = sx_ref[0] * sw_ref[0]
        out_ref[...] = jnp.maximum(out_ref[...] * scale, 0.0)

        for r in sends:
            r.wait_send()

    return pl.pallas_call(
        body,
        out_shape=jax.ShapeDtypeStruct((BM, n), jnp.float32),
        in_specs=[
            pl.BlockSpec(memory_space=pltpu.VMEM),
            pl.BlockSpec(memory_space=pltpu.MemorySpace.HBM),
            pl.BlockSpec(memory_space=pltpu.SMEM),
            pl.BlockSpec(memory_space=pltpu.SMEM),
        ],
        out_specs=pl.BlockSpec(memory_space=pltpu.VMEM),
        scratch_shapes=[
            pltpu.VMEM((m_total, k_shard), COMM_DTYPE),
            pltpu.VMEM((BM, k_total), COMM_DTYPE),
            pltpu.VMEM((2, KC, n), jnp.float32),
            pltpu.SemaphoreType.DMA((2,)),
            pltpu.SemaphoreType.DMA((N_DEV,)),
            pltpu.SemaphoreType.DMA((N_DEV,)),
        ],
        compiler_params=pltpu.CompilerParams(
            vmem_limit_bytes=64 * 1024 * 1024,
        ),
    )(x, w_mat, scale_x, scale_w)


# device time: 62479 ns/iter; 1.0307x vs baseline; 1.0090x over previous
import jax
import jax.numpy as jnp
from jax import lax
from jax.experimental import pallas as pl
from jax.experimental.pallas import tpu as pltpu

N_DEV = 32
BM = 128
KC = 256
NBUF = 4
COMM_DTYPE = jnp.float8_e4m3fn


def kernel(x, w_mat, scale_x, scale_w):
    m_total, k_shard = x.shape
    k_total, n = w_mat.shape
    n_chunks = k_total // KC
    blocks_per_chunk = KC // BM

    def body(x_ref, w_hbm, sx_ref, sw_ref, out_ref,
             x8_ref, xg_ref, wblk_ref, wsems, send_sems, recv_sems):
        my = lax.axis_index("i")

        x8_ref[...] = x_ref[...].astype(COMM_DTYPE)

        xg_ref[:, pl.ds(my * BM, BM)] = x8_ref[pl.ds(my * BM, BM), :]

        sends = []
        for off in range(1, N_DEV):
            peer = lax.rem(my + off, N_DEV)
            rdma = pltpu.make_async_remote_copy(
                src_ref=x8_ref.at[pl.ds(peer * BM, BM), :],
                dst_ref=xg_ref.at[:, pl.ds(my * BM, BM)],
                send_sem=send_sems.at[off],
                recv_sem=recv_sems.at[off],
                device_id=(peer,),
                device_id_type=pl.DeviceIdType.MESH,
            )
            rdma.start()
            sends.append(rdma)

        def start_w(c):
            cp = pltpu.make_async_copy(
                w_hbm.at[pl.ds(c * KC, KC), :],
                wblk_ref.at[c % NBUF],
                wsems.at[c % NBUF],
            )
            cp.start()
            return cp

        w_descs = {c: start_w(c) for c in range(NBUF)}

        for off in range(1, N_DEV):
            s = lax.rem(my - off + N_DEV, N_DEV)
            recv = pltpu.make_async_remote_copy(
                src_ref=x8_ref.at[pl.ds(0, BM), :],
                dst_ref=xg_ref.at[:, pl.ds(s * BM, BM)],
                send_sem=send_sems.at[off],
                recv_sem=recv_sems.at[off],
                device_id=(0,),
                device_id_type=pl.DeviceIdType.MESH,
            )
            recv.wait_recv()

        for c in range(n_chunks):
            w_descs[c % NBUF].wait()
            a = xg_ref[:, pl.ds(c * KC, KC)]
            wb = wblk_ref[c % NBUF].astype(COMM_DTYPE)
            contrib = jnp.dot(a, wb, preferred_element_type=jnp.float32)
            if c == 0:
                out_ref[...] = contrib
            else:
                out_ref[...] += contrib
            if c + NBUF < n_chunks:
                w_descs[c % NBUF] = start_w(c + NBUF)

        scale = sx_ref[0] * sw_ref[0]
        out_ref[...] = jnp.maximum(out_ref[...] * scale, 0.0)

        for r in sends:
            r.wait_send()

    return pl.pallas_call(
        body,
        out_shape=jax.ShapeDtypeStruct((BM, n), jnp.float32),
        in_specs=[
            pl.BlockSpec(memory_space=pltpu.VMEM),
            pl.BlockSpec(memory_space=pltpu.MemorySpace.HBM),
            pl.BlockSpec(memory_space=pltpu.SMEM),
            pl.BlockSpec(memory_space=pltpu.SMEM),
        ],
        out_specs=pl.BlockSpec(memory_space=pltpu.VMEM),
        scratch_shapes=[
            pltpu.VMEM((m_total, k_shard), COMM_DTYPE),
            pltpu.VMEM((BM, k_total), COMM_DTYPE),
            pltpu.VMEM((NBUF, KC, n), jnp.float32),
            pltpu.SemaphoreType.DMA((NBUF,)),
            pltpu.SemaphoreType.DMA((N_DEV,)),
            pltpu.SemaphoreType.DMA((N_DEV,)),
        ],
        compiler_params=pltpu.CompilerParams(
            vmem_limit_bytes=64 * 1024 * 1024,
        ),
    )(x, w_mat, scale_x, scale_w)
